# baseline (speedup 1.0000x reference)
import functools

import jax
import jax.numpy as jnp
from jax.experimental import pallas as pl
from jax.experimental.pallas import tpu as pltpu


def _gcn_body(x_ref, adj_ref, w1_ref, b1_ref, w2_ref, b2_ref,
              feat_ref, out_ref, support_ref, *, half_rows):
    s = pl.program_id(0)

    @pl.when(s == 0)
    def _compute_support():
        support_ref[...] = (
            jnp.dot(x_ref[...].astype(jnp.bfloat16),
                    w1_ref[...].astype(jnp.bfloat16),
                    preferred_element_type=jnp.float32)
            + b1_ref[...]
        ).astype(jnp.bfloat16)

    def _do(adj_half):
        h = jnp.dot(adj_half.astype(jnp.bfloat16), support_ref[...],
                    preferred_element_type=jnp.float32)
        feat = jnp.maximum(h, 0.0)
        feat_ref[...] = feat
        out_ref[...] = jax.nn.sigmoid(
            jnp.dot(feat.astype(jnp.bfloat16), w2_ref[...].astype(jnp.bfloat16),
                    preferred_element_type=jnp.float32)
            + b2_ref[...]
        )

    @pl.when(s % 2 == 0)
    def _lower():
        _do(adj_ref[:half_rows, :])

    @pl.when(s % 2 == 1)
    def _upper():
        _do(adj_ref[half_rows:, :])


@functools.partial(jax.jit, static_argnames=("block_n",))
def _gcn_fused(x, adj, W1, b1, W2, b2, block_n=1024):
    n, f = x.shape
    h_dim = W1.shape[1]
    c = W2.shape[1]
    half = block_n // 2
    b1r = b1.reshape(1, h_dim)
    b2r = b2.reshape(1, c)
    feature, out = pl.pallas_call(
        functools.partial(_gcn_body, half_rows=half),
        grid=(2 * (n // block_n),),
        in_specs=[
            pl.BlockSpec((n, f), lambda s: (0, 0)),
            pl.BlockSpec((block_n, n), lambda s: (s // 2, 0)),
            pl.BlockSpec((f, h_dim), lambda s: (0, 0)),
            pl.BlockSpec((1, h_dim), lambda s: (0, 0)),
            pl.BlockSpec((h_dim, c), lambda s: (0, 0)),
            pl.BlockSpec((1, c), lambda s: (0, 0)),
        ],
        out_specs=[
            pl.BlockSpec((half, h_dim), lambda s: (s, 0)),
            pl.BlockSpec((half, c), lambda s: (s, 0)),
        ],
        out_shape=[
            jax.ShapeDtypeStruct((n, h_dim), jnp.float32),
            jax.ShapeDtypeStruct((n, c), jnp.float32),
        ],
        scratch_shapes=[pltpu.VMEM((n, h_dim), jnp.bfloat16)],
    )(x, adj, W1, b1r, W2, b2r)
    return feature, out


def kernel(x, adj, W1, b1, W2, b2):
    return _gcn_fused(x, adj, W1, b1, W2, b2)
